# baseline (speedup 1.0000x reference)
import functools

import jax
import jax.numpy as jnp
from jax import lax
from jax.experimental import pallas as pl
from jax.experimental.pallas import tpu as pltpu
from jax.experimental.pallas import tpu_sc as plsc

NC = 2
NS = 16
NW = NC * NS

B = 4096
V = 8192
C = 32
H = 16
W = 16
F = C * H * W

SPLITS = 2
D = F // SPLITS
CK = 8
BPW = B // NW
G = BPW // CK
G2 = G // 2


def _gather_body(idx_hbm, table_hbm, out_hbm,
                 idx_v, buf0, buf1, sg0, sg1, ss0, ss1):
    wid = lax.axis_index("s") * NC + lax.axis_index("c")
    base = wid * BPW
    pltpu.sync_copy(idx_hbm.at[wid], idx_v)

    def gather(g, buf, sem):
        return pltpu.make_async_copy(table_hbm.at[idx_v.at[g]], buf, sem)

    def store(g, buf, sem):
        return pltpu.make_async_copy(
            buf, out_hbm.at[pl.ds(base + g * CK, CK)], sem)

    gather(0, buf0, sg0).start()

    def body(p, carry):
        g0 = 2 * p
        g1 = g0 + 1

        @pl.when(p > 0)
        def _():
            store(g1 - 2, buf1, ss1).wait()

        gather(g1, buf1, sg1).start()
        gather(g0, buf0, sg0).wait()
        store(g0, buf0, ss0).start()

        @pl.when(p < G2 - 1)
        def _():
            store(g0, buf0, ss0).wait()
            gather(g0 + 2, buf0, sg0).start()

        gather(g1, buf1, sg1).wait()
        store(g1, buf1, ss1).start()
        return carry

    lax.fori_loop(0, G2, body, 0, unroll=False)
    store(G - 2, buf0, ss0).wait()
    store(G - 1, buf1, ss1).wait()


def _gather_slice(idx3, table):
    mesh = plsc.VectorSubcoreMesh(
        core_axis_name="c", subcore_axis_name="s", num_cores=NC, num_subcores=NS
    )
    return pl.kernel(
        _gather_body,
        out_type=jax.ShapeDtypeStruct((B, D), jnp.float32),
        mesh=mesh,
        scratch_types=[
            pltpu.VMEM((G, CK), jnp.int32),
            pltpu.VMEM((CK, D), jnp.float32),
            pltpu.VMEM((CK, D), jnp.float32),
            pltpu.SemaphoreType.DMA,
            pltpu.SemaphoreType.DMA,
            pltpu.SemaphoreType.DMA,
            pltpu.SemaphoreType.DMA,
        ],
    )(idx3, table)


@jax.jit
def _run(idx, code):
    idx3 = idx.astype(jnp.int32).reshape(NW, G, CK)
    cs = C // SPLITS
    parts = []
    for h in range(SPLITS):
        table_h = code[:, h * cs:(h + 1) * cs].reshape(V, D)
        out_h = _gather_slice(idx3, table_h)
        parts.append(out_h.reshape(B, cs, H, W))
    return jnp.concatenate(parts, axis=1)


def kernel(idx, code):
    return _run(idx, code)


# speedup vs baseline: 1.2639x; 1.2639x over previous
import functools

import jax
import jax.numpy as jnp
from jax import lax
from jax.experimental import pallas as pl
from jax.experimental.pallas import tpu as pltpu
from jax.experimental.pallas import tpu_sc as plsc

NC = 2
NS = 16
NW = NC * NS

B = 4096
V = 8192
C = 32
H = 16
W = 16
F = C * H * W

CK = 4
BPW = B // NW
G = BPW // CK
G2 = G // 2

TT = 512
TC_BLK = TT // (H * W)


def _tp_in_body(x_ref, o_ref):
    x = x_ref[...].reshape(TT, TT)
    o_ref[...] = x.T


def _tp_table(code_fm):
    return pl.pallas_call(
        _tp_in_body,
        grid=(F // TT, V // TT),
        in_specs=[pl.BlockSpec((TC_BLK, H, W, TT), lambda i, j: (i, 0, 0, j))],
        out_specs=pl.BlockSpec((TT, TT), lambda i, j: (j, i)),
        out_shape=jax.ShapeDtypeStruct((V, F), jnp.float32),
    )(code_fm)


def _tp_out_body(x_ref, o_ref):
    x = x_ref[...]
    o_ref[...] = x.T.reshape(TC_BLK, H, W, TT)


def _tp_out(out_rm):
    return pl.pallas_call(
        _tp_out_body,
        grid=(F // TT, B // TT),
        in_specs=[pl.BlockSpec((TT, TT), lambda i, j: (j, i))],
        out_specs=pl.BlockSpec((TC_BLK, H, W, TT), lambda i, j: (i, 0, 0, j)),
        out_shape=jax.ShapeDtypeStruct((C, H, W, B), jnp.float32),
    )(out_rm)


def _gather_body(idx_hbm, table_hbm, out_hbm,
                 idx_v, buf0, buf1, sg0, sg1, ss0, ss1):
    wid = lax.axis_index("s") * NC + lax.axis_index("c")
    base = wid * BPW
    pltpu.sync_copy(idx_hbm.at[wid], idx_v)

    def gather(g, buf, sem):
        return pltpu.make_async_copy(table_hbm.at[idx_v.at[g]], buf, sem)

    def store(g, buf, sem):
        return pltpu.make_async_copy(
            buf, out_hbm.at[pl.ds(base + g * CK, CK)], sem)

    gather(0, buf0, sg0).start()

    def body(p, carry):
        g0 = 2 * p
        g1 = g0 + 1

        @pl.when(p > 0)
        def _():
            store(g1 - 2, buf1, ss1).wait()

        gather(g1, buf1, sg1).start()
        gather(g0, buf0, sg0).wait()
        store(g0, buf0, ss0).start()

        @pl.when(p < G2 - 1)
        def _():
            store(g0, buf0, ss0).wait()
            gather(g0 + 2, buf0, sg0).start()

        gather(g1, buf1, sg1).wait()
        store(g1, buf1, ss1).start()
        return carry

    lax.fori_loop(0, G2, body, 0, unroll=False)
    store(G - 2, buf0, ss0).wait()
    store(G - 1, buf1, ss1).wait()


def _gather(idx3, table):
    mesh = plsc.VectorSubcoreMesh(
        core_axis_name="c", subcore_axis_name="s", num_cores=NC, num_subcores=NS
    )
    return pl.kernel(
        _gather_body,
        out_type=jax.ShapeDtypeStruct((B, F), jnp.float32),
        mesh=mesh,
        scratch_types=[
            pltpu.VMEM((G, CK), jnp.int32),
            pltpu.VMEM((CK, F), jnp.float32),
            pltpu.VMEM((CK, F), jnp.float32),
            pltpu.SemaphoreType.DMA,
            pltpu.SemaphoreType.DMA,
            pltpu.SemaphoreType.DMA,
            pltpu.SemaphoreType.DMA,
        ],
    )(idx3, table)


@jax.jit
def _run(idx, code):
    idx3 = idx.astype(jnp.int32).reshape(NW, G, CK)
    code_fm = code.transpose(1, 2, 3, 0)
    table_rm = _tp_table(code_fm)
    out_rm = _gather(idx3, table_rm)
    out_fm = _tp_out(out_rm)
    return out_fm.transpose(3, 0, 1, 2)


def kernel(idx, code):
    return _run(idx, code)


# speedup vs baseline: 1.8434x; 1.4584x over previous
import functools

import jax
import jax.numpy as jnp
from jax import lax
from jax.experimental import pallas as pl
from jax.experimental.pallas import tpu as pltpu
from jax.experimental.pallas import tpu_sc as plsc

NC = 2
NS = 16
NW = NC * NS

B = 4096
V = 8192
C = 32
H = 16
W = 16
F = C * H * W

CK = 4
BPW = B // NW
G = BPW // CK
G2 = G // 2

TT = 1024
TC_BLK = TT // (H * W)


def _tp_in_body(x_ref, o_ref):
    x = x_ref[...].reshape(TT, TT)
    o_ref[...] = x.T


def _tp_table(code_fm):
    return pl.pallas_call(
        _tp_in_body,
        grid=(F // TT, V // TT),
        in_specs=[pl.BlockSpec((TC_BLK, H, W, TT), lambda i, j: (i, 0, 0, j))],
        out_specs=pl.BlockSpec((TT, TT), lambda i, j: (j, i)),
        out_shape=jax.ShapeDtypeStruct((V, F), jnp.float32),
    )(code_fm)


def _tp_out_body(x_ref, o_ref):
    x = x_ref[...]
    o_ref[...] = x.T.reshape(TC_BLK, H, W, TT)


def _tp_out(out_rm):
    return pl.pallas_call(
        _tp_out_body,
        grid=(F // TT, B // TT),
        in_specs=[pl.BlockSpec((TT, TT), lambda i, j: (j, i))],
        out_specs=pl.BlockSpec((TC_BLK, H, W, TT), lambda i, j: (i, 0, 0, j)),
        out_shape=jax.ShapeDtypeStruct((C, H, W, B), jnp.float32),
    )(out_rm)


def _gather_body(idx_hbm, table_hbm, out_hbm,
                 idx_v, buf0, buf1, sg0, sg1, ss0, ss1):
    wid = lax.axis_index("s") * NC + lax.axis_index("c")
    base = wid * BPW
    pltpu.sync_copy(idx_hbm.at[wid], idx_v)

    def gather(g, buf, sem):
        return pltpu.make_async_copy(table_hbm.at[idx_v.at[g]], buf, sem)

    def store(g, buf, sem):
        return pltpu.make_async_copy(
            buf, out_hbm.at[pl.ds(base + g * CK, CK)], sem)

    gather(0, buf0, sg0).start()

    def body(p, carry):
        g0 = 2 * p
        g1 = g0 + 1

        @pl.when(p > 0)
        def _():
            store(g1 - 2, buf1, ss1).wait()

        gather(g1, buf1, sg1).start()
        gather(g0, buf0, sg0).wait()
        store(g0, buf0, ss0).start()

        @pl.when(p < G2 - 1)
        def _():
            store(g0, buf0, ss0).wait()
            gather(g0 + 2, buf0, sg0).start()

        gather(g1, buf1, sg1).wait()
        store(g1, buf1, ss1).start()
        return carry

    lax.fori_loop(0, G2, body, 0, unroll=False)
    store(G - 2, buf0, ss0).wait()
    store(G - 1, buf1, ss1).wait()


def _gather(idx3, table):
    mesh = plsc.VectorSubcoreMesh(
        core_axis_name="c", subcore_axis_name="s", num_cores=NC, num_subcores=NS
    )
    return pl.kernel(
        _gather_body,
        out_type=jax.ShapeDtypeStruct((B, F), jnp.float32),
        mesh=mesh,
        scratch_types=[
            pltpu.VMEM((G, CK), jnp.int32),
            pltpu.VMEM((CK, F), jnp.float32),
            pltpu.VMEM((CK, F), jnp.float32),
            pltpu.SemaphoreType.DMA,
            pltpu.SemaphoreType.DMA,
            pltpu.SemaphoreType.DMA,
            pltpu.SemaphoreType.DMA,
        ],
    )(idx3, table)


@jax.jit
def _run(idx, code):
    idx3 = idx.astype(jnp.int32).reshape(NW, G, CK)
    code_fm = code.transpose(1, 2, 3, 0)
    table_rm = _tp_table(code_fm)
    out_rm = _gather(idx3, table_rm)
    out_fm = _tp_out(out_rm)
    return out_fm.transpose(3, 0, 1, 2)


def kernel(idx, code):
    return _run(idx, code)


# speedup vs baseline: 1.8666x; 1.0126x over previous
import functools

import jax
import jax.numpy as jnp
from jax import lax
from jax.experimental import pallas as pl
from jax.experimental.pallas import tpu as pltpu
from jax.experimental.pallas import tpu_sc as plsc

NC = 2
NS = 16
NW = NC * NS

B = 4096
V = 8192
C = 32
H = 16
W = 16
F = C * H * W

CK = 4
BPW = B // NW
G = BPW // CK
G2 = G // 2

TT = 1024
TC_BLK = TT // (H * W)

TTF = 1024
TTV = 2048
TCF_BLK = TTF // (H * W)


def _tp_in_body(x_ref, o_ref):
    x = x_ref[...].reshape(TTF, TTV)
    o_ref[...] = x.T


def _tp_table(code_fm):
    return pl.pallas_call(
        _tp_in_body,
        grid=(F // TTF, V // TTV),
        in_specs=[pl.BlockSpec((TCF_BLK, H, W, TTV), lambda i, j: (i, 0, 0, j))],
        out_specs=pl.BlockSpec((TTV, TTF), lambda i, j: (j, i)),
        out_shape=jax.ShapeDtypeStruct((V, F), jnp.float32),
    )(code_fm)


def _tp_out_body(x_ref, o_ref):
    x = x_ref[...]
    o_ref[...] = x.T.reshape(TC_BLK, H, W, TT)


def _tp_out(out_rm):
    return pl.pallas_call(
        _tp_out_body,
        grid=(F // TT, B // TT),
        in_specs=[pl.BlockSpec((TT, TT), lambda i, j: (j, i))],
        out_specs=pl.BlockSpec((TC_BLK, H, W, TT), lambda i, j: (i, 0, 0, j)),
        out_shape=jax.ShapeDtypeStruct((C, H, W, B), jnp.float32),
    )(out_rm)


def _gather_body(idx_hbm, table_hbm, out_hbm,
                 idx_v, buf0, buf1, sg0, sg1, ss0, ss1):
    wid = lax.axis_index("s") * NC + lax.axis_index("c")
    base = wid * BPW
    pltpu.sync_copy(idx_hbm.at[wid], idx_v)

    def gather(g, buf, sem):
        return pltpu.make_async_copy(table_hbm.at[idx_v.at[g]], buf, sem)

    def store(g, buf, sem):
        return pltpu.make_async_copy(
            buf, out_hbm.at[pl.ds(base + g * CK, CK)], sem)

    gather(0, buf0, sg0).start()

    def body(p, carry):
        g0 = 2 * p
        g1 = g0 + 1

        @pl.when(p > 0)
        def _():
            store(g1 - 2, buf1, ss1).wait()

        gather(g1, buf1, sg1).start()
        gather(g0, buf0, sg0).wait()
        store(g0, buf0, ss0).start()

        @pl.when(p < G2 - 1)
        def _():
            store(g0, buf0, ss0).wait()
            gather(g0 + 2, buf0, sg0).start()

        gather(g1, buf1, sg1).wait()
        store(g1, buf1, ss1).start()
        return carry

    lax.fori_loop(0, G2, body, 0, unroll=False)
    store(G - 2, buf0, ss0).wait()
    store(G - 1, buf1, ss1).wait()


def _gather(idx3, table):
    mesh = plsc.VectorSubcoreMesh(
        core_axis_name="c", subcore_axis_name="s", num_cores=NC, num_subcores=NS
    )
    return pl.kernel(
        _gather_body,
        out_type=jax.ShapeDtypeStruct((B, F), jnp.float32),
        mesh=mesh,
        scratch_types=[
            pltpu.VMEM((G, CK), jnp.int32),
            pltpu.VMEM((CK, F), jnp.float32),
            pltpu.VMEM((CK, F), jnp.float32),
            pltpu.SemaphoreType.DMA,
            pltpu.SemaphoreType.DMA,
            pltpu.SemaphoreType.DMA,
            pltpu.SemaphoreType.DMA,
        ],
    )(idx3, table)


@jax.jit
def _run(idx, code):
    idx3 = idx.astype(jnp.int32).reshape(NW, G, CK)
    code_fm = code.transpose(1, 2, 3, 0)
    table_rm = _tp_table(code_fm)
    out_rm = _gather(idx3, table_rm)
    out_fm = _tp_out(out_rm)
    return out_fm.transpose(3, 0, 1, 2)


def kernel(idx, code):
    return _run(idx, code)


# speedup vs baseline: 1.8840x; 1.0093x over previous
import functools

import jax
import jax.numpy as jnp
from jax import lax
from jax.experimental import pallas as pl
from jax.experimental.pallas import tpu as pltpu
from jax.experimental.pallas import tpu_sc as plsc

NC = 2
NS = 16
NW = NC * NS

B = 4096
V = 8192
C = 32
H = 16
W = 16
F = C * H * W

CK = 4
BPW = B // NW
G = BPW // CK
G2 = G // 2

TT = 1024
TC_BLK = TT // (H * W)

TTF = 1024
TTV = 2048
TCF_BLK = TTF // (H * W)


def _tp_in_body(x_ref, o_ref):
    x = x_ref[...].reshape(TTF, TTV)
    o_ref[...] = x.T


def _tp_table(code_fm):
    return pl.pallas_call(
        _tp_in_body,
        grid=(F // TTF, V // TTV),
        in_specs=[pl.BlockSpec((TCF_BLK, H, W, TTV), lambda i, j: (i, 0, 0, j))],
        out_specs=pl.BlockSpec((TTV, TTF), lambda i, j: (j, i)),
        out_shape=jax.ShapeDtypeStruct((V, F), jnp.float32),
    )(code_fm)


TTB = 2048


def _tp_out_body(x_ref, o_ref):
    x = x_ref[...]
    o_ref[...] = x.T.reshape(TC_BLK, H, W, TTB)


def _tp_out(out_rm):
    return pl.pallas_call(
        _tp_out_body,
        grid=(F // TT, B // TTB),
        in_specs=[pl.BlockSpec((TTB, TT), lambda i, j: (j, i))],
        out_specs=pl.BlockSpec((TC_BLK, H, W, TTB), lambda i, j: (i, 0, 0, j)),
        out_shape=jax.ShapeDtypeStruct((C, H, W, B), jnp.float32),
    )(out_rm)


def _gather_body(idx_hbm, table_hbm, out_hbm,
                 idx_v, buf0, buf1, sg0, sg1, ss0, ss1):
    wid = lax.axis_index("s") * NC + lax.axis_index("c")
    base = wid * BPW
    pltpu.sync_copy(idx_hbm.at[wid], idx_v)

    def gather(g, buf, sem):
        return pltpu.make_async_copy(table_hbm.at[idx_v.at[g]], buf, sem)

    def store(g, buf, sem):
        return pltpu.make_async_copy(
            buf, out_hbm.at[pl.ds(base + g * CK, CK)], sem)

    gather(0, buf0, sg0).start()

    def body(p, carry):
        g0 = 2 * p
        g1 = g0 + 1

        @pl.when(p > 0)
        def _():
            store(g1 - 2, buf1, ss1).wait()

        gather(g1, buf1, sg1).start()
        gather(g0, buf0, sg0).wait()
        store(g0, buf0, ss0).start()

        @pl.when(p < G2 - 1)
        def _():
            store(g0, buf0, ss0).wait()
            gather(g0 + 2, buf0, sg0).start()

        gather(g1, buf1, sg1).wait()
        store(g1, buf1, ss1).start()
        return carry

    lax.fori_loop(0, G2, body, 0, unroll=False)
    store(G - 2, buf0, ss0).wait()
    store(G - 1, buf1, ss1).wait()


def _gather(idx3, table):
    mesh = plsc.VectorSubcoreMesh(
        core_axis_name="c", subcore_axis_name="s", num_cores=NC, num_subcores=NS
    )
    return pl.kernel(
        _gather_body,
        out_type=jax.ShapeDtypeStruct((B, F), jnp.float32),
        mesh=mesh,
        scratch_types=[
            pltpu.VMEM((G, CK), jnp.int32),
            pltpu.VMEM((CK, F), jnp.float32),
            pltpu.VMEM((CK, F), jnp.float32),
            pltpu.SemaphoreType.DMA,
            pltpu.SemaphoreType.DMA,
            pltpu.SemaphoreType.DMA,
            pltpu.SemaphoreType.DMA,
        ],
    )(idx3, table)


@jax.jit
def _run(idx, code):
    idx3 = idx.astype(jnp.int32).reshape(NW, G, CK)
    code_fm = code.transpose(1, 2, 3, 0)
    table_rm = _tp_table(code_fm)
    out_rm = _gather(idx3, table_rm)
    out_fm = _tp_out(out_rm)
    return out_fm.transpose(3, 0, 1, 2)


def kernel(idx, code):
    return _run(idx, code)
